# Initial kernel scaffold; baseline (speedup 1.0000x reference)
import jax
import jax.numpy as jnp
from jax.experimental import pallas as pl


def kernel(x, W, W1, b1, W2, b2):
    raise NotImplementedError("write your pallas kernel here")


# speedup vs baseline: 27.7576x; 27.7576x over previous
import functools

import jax
import jax.numpy as jnp
from jax import lax
from jax.experimental import pallas as pl
from jax.experimental.pallas import tpu as pltpu
from jax.experimental.pallas import tpu_sc as plsc

_VOCAB = 100000
_EMB = 50
_HID = 32
_BATCH = 16384
_SEQ = 200

_NC = 2
_NS = 16
_NW = _NC * _NS
_BPW = _BATCH // _NW

_CHUNK_A = 104
_CHUNK_B = _SEQ - _CHUNK_A

_VBLK = 1000


def _table_body(w_ref, w1_ref, t_ref):
    t_ref[...] = jnp.dot(
        w_ref[...], w1_ref[...], preferred_element_type=jnp.float32
    ) * (1.0 / _SEQ)


def _fused_table(w, w1):
    return pl.pallas_call(
        _table_body,
        grid=(_VOCAB // _VBLK,),
        in_specs=[
            pl.BlockSpec((_VBLK, _EMB), lambda i: (i, 0)),
            pl.BlockSpec((_EMB, _HID), lambda i: (0, 0)),
        ],
        out_specs=pl.BlockSpec((_VBLK, _HID), lambda i: (i, 0)),
        out_shape=jax.ShapeDtypeStruct((_VOCAB, _HID), jnp.float32),
    )(w, w1)


def _sc_body(t_hbm, x_hbm, p_hbm, out_hbm,
             idx_v, rows0, rows1, out_v, p_v, sem0, sem1):
    wid = lax.axis_index("s") * _NC + lax.axis_index("c")
    base = wid * _BPW

    pltpu.sync_copy(x_hbm.at[pl.ds(base * _SEQ, _BPW * _SEQ)], idx_v)
    pltpu.sync_copy(p_hbm, p_v)

    bufs = (rows0, rows1)
    sems = (sem0, sem1)

    def issue(b, j):
        pltpu.async_copy(
            t_hbm.at[idx_v.at[pl.ds(b * _SEQ, _CHUNK_A)]],
            bufs[j].at[pl.ds(0, _CHUNK_A)], sems[j])
        pltpu.async_copy(
            t_hbm.at[idx_v.at[pl.ds(b * _SEQ + _CHUNK_A, _CHUNK_B)]],
            bufs[j].at[pl.ds(_CHUNK_A, _CHUNK_B)], sems[j])

    def wait(b, j):
        pltpu.make_async_copy(
            t_hbm.at[idx_v.at[pl.ds(b * _SEQ, _CHUNK_A)]],
            bufs[j].at[pl.ds(0, _CHUNK_A)], sems[j]).wait()
        pltpu.make_async_copy(
            t_hbm.at[idx_v.at[pl.ds(b * _SEQ + _CHUNK_A, _CHUNK_B)]],
            bufs[j].at[pl.ds(_CHUNK_A, _CHUNK_B)], sems[j]).wait()

    b1a = p_v[pl.ds(0, 16)]
    b1b = p_v[pl.ds(16, 16)]
    w2a = p_v[pl.ds(32, 16)]
    w2b = p_v[pl.ds(48, 16)]
    b2s = p_v[pl.ds(64, 16)][0]
    lane = lax.iota(jnp.int32, 16)

    issue(0, 0)
    issue(1, 1)

    unroll = 8

    def outer(g, ovec):
        for j in range(2):
            b = 2 * g + j
            wait(b, j)
            rows = bufs[j]

            def acc_body(i, accs):
                a0, a1 = accs
                for k in range(unroll):
                    r = i * unroll + k
                    a0 = a0 + rows[r, pl.ds(0, 16)]
                    a1 = a1 + rows[r, pl.ds(16, 16)]
                return (a0, a1)

            z = jnp.zeros((16,), jnp.float32)
            a0, a1 = lax.fori_loop(0, _SEQ // unroll, acc_body, (z, z))

            h0 = jnp.maximum(a0 + b1a, 0.0) * w2a
            h1 = jnp.maximum(a1 + b1b, 0.0) * w2b
            s = jnp.sum(h0 + h1) + b2s
            ovec = jnp.where(lane == (b & 15), s, ovec)

            @pl.when((b & 15) == 15)
            def _():
                out_v[pl.ds(b - 15, 16)] = ovec

            nb = b + 2

            @pl.when(nb < _BPW)
            def _():
                issue(nb, j)
        return ovec

    pl.loop(0, _BPW // 2, init_carry=jnp.zeros((16,), jnp.float32))(outer)

    pltpu.sync_copy(out_v, out_hbm.at[pl.ds(base, _BPW)])


_sc_call = functools.partial(
    pl.kernel,
    mesh=plsc.VectorSubcoreMesh(core_axis_name="c", subcore_axis_name="s"),
    out_type=jax.ShapeDtypeStruct((_BATCH,), jnp.float32),
    scratch_types=[
        pltpu.VMEM((_BPW * _SEQ,), jnp.int32),
        pltpu.VMEM((_SEQ, _HID), jnp.float32),
        pltpu.VMEM((_SEQ, _HID), jnp.float32),
        pltpu.VMEM((_BPW,), jnp.float32),
        pltpu.VMEM((80,), jnp.float32),
        pltpu.SemaphoreType.DMA,
        pltpu.SemaphoreType.DMA,
    ],
    compiler_params=pltpu.CompilerParams(
        needs_layout_passes=False, use_tc_tiling_on_sc=False),
)


@jax.jit
def kernel(x, W, W1, b1, W2, b2):
    t = _fused_table(W, W1)
    params = jnp.concatenate(
        [b1, W2[:, 0], jnp.full((16,), b2[0], jnp.float32)])
    out = _sc_call(_sc_body)(t, x.astype(jnp.int32).reshape(-1), params)
    return out.reshape(_BATCH, 1)
